# baseline (speedup 1.0000x reference)
import jax
import jax.numpy as jnp
from jax.experimental import pallas as pl
from jax.experimental.pallas import tpu as pltpu

_BLK = 4096
_P = 14


def _make_kernel(nblk, tail):
    def body(x_ref, b_ref, w_hbm, o_hbm,
             wbuf, wtail, obuf, otail, wsem, wtsem, osem, otsem):
        s = pl.program_id(0)

        @pl.when(s == 0)
        def _():
            for k in range(_P):
                pltpu.make_async_copy(
                    w_hbm.at[pl.ds(k * _BLK, _BLK), :],
                    wbuf.at[k],
                    wsem.at[k],
                ).start()

        for k in range(_P):
            blkid = s * _P + k

            @pl.when(blkid < nblk - 1)
            def _(k=k, blkid=blkid):
                pltpu.make_async_copy(
                    w_hbm.at[pl.ds(blkid * _BLK, _BLK), :],
                    wbuf.at[k],
                    wsem.at[k],
                ).wait()
                obuf[k] = jax.lax.dot_general(
                    x_ref[...], wbuf[k],
                    (((1,), (1,)), ((), ())),
                    preferred_element_type=jnp.float32,
                ) + b_ref[:, k * _BLK:(k + 1) * _BLK]

            @pl.when(blkid == nblk - 1)
            def _(k=k):
                pltpu.make_async_copy(
                    w_hbm.at[pl.ds((nblk - 1) * _BLK, tail), :],
                    wtail,
                    wtsem,
                ).wait()
                otail[...] = jax.lax.dot_general(
                    x_ref[...], wtail[...],
                    (((1,), (1,)), ((), ())),
                    preferred_element_type=jnp.float32,
                ) + b_ref[:, k * _BLK:k * _BLK + tail]

        for k in range(_P):
            blkid = s * _P + k

            @pl.when(blkid < nblk - 1)
            def _(k=k, blkid=blkid):
                pltpu.make_async_copy(
                    obuf.at[k],
                    o_hbm.at[:, pl.ds(blkid * _BLK, _BLK)],
                    osem.at[k],
                ).start()

            @pl.when(blkid == nblk - 1)
            def _():
                pltpu.make_async_copy(
                    otail,
                    o_hbm.at[:, pl.ds((nblk - 1) * _BLK, tail)],
                    otsem,
                ).start()

        for k in range(_P):
            blkid = s * _P + k

            @pl.when(blkid < nblk - 1)
            def _(k=k, blkid=blkid):
                pltpu.make_async_copy(
                    obuf.at[k],
                    o_hbm.at[:, pl.ds(blkid * _BLK, _BLK)],
                    osem.at[k],
                ).wait()

            @pl.when(blkid == nblk - 1)
            def _():
                pltpu.make_async_copy(
                    otail,
                    o_hbm.at[:, pl.ds((nblk - 1) * _BLK, tail)],
                    otsem,
                ).wait()

        for k in range(_P):
            nb = (s + 1) * _P + k

            @pl.when(nb < nblk - 1)
            def _(k=k, nb=nb):
                pltpu.make_async_copy(
                    w_hbm.at[pl.ds(nb * _BLK, _BLK), :],
                    wbuf.at[k],
                    wsem.at[k],
                ).start()

            @pl.when(nb == nblk - 1)
            def _():
                pltpu.make_async_copy(
                    w_hbm.at[pl.ds((nblk - 1) * _BLK, tail), :],
                    wtail,
                    wtsem,
                ).start()

    return body


def kernel(x, y, freeze_flag, W, b):
    del y, freeze_flag
    Bm, D = x.shape
    C1 = W.shape[0]
    nblk = pl.cdiv(C1, _BLK)
    tail = C1 - (nblk - 1) * _BLK
    nsuper = pl.cdiv(nblk, _P)
    b2 = b.reshape(1, C1)
    out = pl.pallas_call(
        _make_kernel(nblk, tail),
        grid=(nsuper,),
        in_specs=[
            pl.BlockSpec((Bm, D), lambda s: (0, 0)),
            pl.BlockSpec((1, _P * _BLK), lambda s: (0, s)),
            pl.BlockSpec(memory_space=pl.ANY),
        ],
        out_specs=pl.BlockSpec(memory_space=pl.ANY),
        out_shape=jax.ShapeDtypeStruct((Bm, C1), jnp.float32),
        scratch_shapes=[
            pltpu.VMEM((_P, _BLK, D), jnp.float32),
            pltpu.VMEM((tail, D), jnp.float32),
            pltpu.VMEM((_P, Bm, _BLK), jnp.float32),
            pltpu.VMEM((Bm, tail), jnp.float32),
            pltpu.SemaphoreType.DMA((_P,)),
            pltpu.SemaphoreType.DMA,
            pltpu.SemaphoreType.DMA((_P,)),
            pltpu.SemaphoreType.DMA,
        ],
        compiler_params=pltpu.CompilerParams(
            dimension_semantics=("arbitrary",),
            vmem_limit_bytes=63 * 1024 * 1024,
        ),
    )(x, b2, W)
    return out
